# baseline (speedup 1.0000x reference)
import functools
import jax
import jax.numpy as jnp
from jax import lax
from jax.experimental import pallas as pl
from jax.experimental.pallas import tpu as pltpu, tpu_sc as plsc

NC = 2
NS = 16
NW = NC * NS


def _build(n_rows, S, H, V):
    C = 80
    GRP = 5
    NBUF = 2 * GRP
    G = 8
    R = n_rows // NW
    assert n_rows % NW == 0 and R % (C * GRP * G) == 0
    n_bodies = R // (C * GRP * G)

    mesh = plsc.VectorSubcoreMesh(core_axis_name="c", subcore_axis_name="s")

    @functools.partial(
        pl.kernel,
        out_type=jax.ShapeDtypeStruct((n_rows, H), jnp.float32),
        mesh=mesh,
        scratch_types=[
            pltpu.VMEM_SHARED((2 * S, H), jnp.float32),
            [pltpu.VMEM((C,), jnp.int32) for _ in range(NBUF)],
            [pltpu.VMEM((C, H), jnp.float32) for _ in range(NBUF)],
            [pltpu.SemaphoreType.DMA for _ in range(2)],
            [pltpu.SemaphoreType.DMA for _ in range(2)],
            [pltpu.SemaphoreType.DMA for _ in range(2)],
            [pltpu.SemaphoreType.DMA for _ in range(2)],
        ],
    )
    def emb_kernel(x_hbm, emb_hbm, pos_hbm, out_hbm, pos2_s, idx, buf,
                   sem_ix, sem_in, sem_g, sem_o):
        sid = lax.axis_index("s")
        wid = sid * NC + lax.axis_index("c")
        base = wid * R

        @pl.when(sid == 0)
        def _():
            pltpu.sync_copy(pos_hbm, pos2_s.at[pl.ds(0, S)])
            pltpu.sync_copy(pos_hbm, pos2_s.at[pl.ds(S, S)])

        plsc.subcore_barrier()

        bank = [list(range(GRP)), list(range(GRP, NBUF))]

        def prep_bank(t0, k):
            descs = []
            for i, b in enumerate(bank[k]):
                t = t0 + i
                a1 = pltpu.async_copy(x_hbm.at[pl.ds(base + t * C, C)],
                                      idx[b], sem_ix[k])
                phase = pl.multiple_of(lax.rem(t * C, S), 8)
                a2 = pltpu.async_copy(pos2_s.at[pl.ds(phase, C)], buf[b],
                                      sem_in[k])
                descs.append((a1, a2))
            return descs

        @pl.loop(0, n_bodies)
        def body(jo):
            j0 = jo * GRP * G
            preps = {0: prep_bank(j0, 0)}
            last_sc = {}
            for g in range(G):
                k = g % 2
                ky = (g + 1) % 2
                t0 = j0 + g * GRP
                for a1, a2 in preps.pop(k):
                    a1.wait()
                    a2.wait()
                gathers = [
                    pltpu.async_copy(emb_hbm.at[idx[b]], buf[b], sem_g[k],
                                     add=True)
                    for b in bank[k]
                ]
                if g + 1 < G:
                    for sc in last_sc.pop(ky, ()):
                        sc.wait()
                    preps[ky] = prep_bank(t0 + GRP, ky)
                for d in gathers:
                    d.wait()
                last_sc[k] = [
                    pltpu.async_copy(
                        buf[b], out_hbm.at[pl.ds(base + (t0 + i) * C, C)],
                        sem_o[k])
                    for i, b in enumerate(bank[k])
                ]
            for kk in sorted(last_sc):
                for sc in last_sc[kk]:
                    sc.wait()

    return emb_kernel


def kernel(x, emb_table, pos_table):
    B, S = x.shape
    V, H = emb_table.shape
    xf = x.reshape(B * S).astype(jnp.int32)
    out = _build(B * S, S, H, V)(xf, emb_table, pos_table)
    return out.reshape(B, S, H)


# speedup vs baseline: 1.0532x; 1.0532x over previous
import functools
import jax
import jax.numpy as jnp
from jax import lax
from jax.experimental import pallas as pl
from jax.experimental.pallas import tpu as pltpu, tpu_sc as plsc

NC = 2
NS = 16
NW = NC * NS


def _build(n_rows, S, H, V):
    C = 128
    NBUF = 5
    W = 10
    R = n_rows // NW
    assert n_rows % NW == 0 and R % (C * NBUF * W) == 0
    n_bodies = R // (C * NBUF * W)

    mesh = plsc.VectorSubcoreMesh(core_axis_name="c", subcore_axis_name="s")

    @functools.partial(
        pl.kernel,
        out_type=jax.ShapeDtypeStruct((n_rows, H), jnp.float32),
        mesh=mesh,
        scratch_types=[
            pltpu.VMEM_SHARED((2 * S, H), jnp.float32),
            [pltpu.VMEM((C,), jnp.int32) for _ in range(NBUF)],
            [pltpu.VMEM((C, H), jnp.float32) for _ in range(NBUF)],
            [pltpu.SemaphoreType.DMA for _ in range(NBUF)],
            [pltpu.SemaphoreType.DMA for _ in range(NBUF)],
            [pltpu.SemaphoreType.DMA for _ in range(NBUF)],
            [pltpu.SemaphoreType.DMA for _ in range(NBUF)],
        ],
    )
    def emb_kernel(x_hbm, emb_hbm, pos_hbm, out_hbm, pos2_s, idx, buf,
                   sem_ix, sem_in, sem_g, sem_o):
        sid = lax.axis_index("s")
        wid = sid * NC + lax.axis_index("c")
        base = wid * R

        @pl.when(sid == 0)
        def _():
            pltpu.sync_copy(pos_hbm, pos2_s.at[pl.ds(0, S)])
            pltpu.sync_copy(pos_hbm, pos2_s.at[pl.ds(S, S)])

        plsc.subcore_barrier()

        def prep(t, b):
            a1 = pltpu.async_copy(x_hbm.at[pl.ds(base + t * C, C)],
                                  idx[b], sem_ix[b])
            phase = pl.multiple_of(lax.rem(t * C, S), 8)
            a2 = pltpu.async_copy(pos2_s.at[pl.ds(phase, C)], buf[b],
                                  sem_in[b])
            return a1, a2

        @pl.loop(0, n_bodies)
        def body(jo):
            j0 = jo * NBUF * W
            preps = [prep(j0 + b, b) for b in range(NBUF)]
            scatters = None
            for w in range(W):
                t0 = j0 + w * NBUF
                gathers = []
                for b in range(NBUF):
                    a1, a2 = preps[b]
                    a1.wait()
                    a2.wait()
                    gathers.append(
                        pltpu.async_copy(emb_hbm.at[idx[b]], buf[b],
                                         sem_g[b], add=True))
                scatters = []
                for b in range(NBUF):
                    gathers[b].wait()
                    scatters.append(
                        pltpu.async_copy(
                            buf[b], out_hbm.at[pl.ds(base + (t0 + b) * C, C)],
                            sem_o[b]))
                if w < W - 1:
                    preps = []
                    for b in range(NBUF):
                        scatters[b].wait()
                        preps.append(prep(t0 + NBUF + b, b))
            for sc in scatters:
                sc.wait()

    return emb_kernel


def kernel(x, emb_table, pos_table):
    B, S = x.shape
    V, H = emb_table.shape
    xf = x.reshape(B * S).astype(jnp.int32)
    out = _build(B * S, S, H, V)(xf, emb_table, pos_table)
    return out.reshape(B, S, H)
